# Initial kernel scaffold; baseline (speedup 1.0000x reference)
import jax
import jax.numpy as jnp
from jax.experimental import pallas as pl


def kernel(feature_global, W1, b1, dir_s, w_t1, b_t1, dir_t1, w_t2, b_t2, dir_t2, w_t3, b_t3, dir_t3):
    raise NotImplementedError("write your pallas kernel here")


# speedup vs baseline: 6.6411x; 6.6411x over previous
import jax
import jax.numpy as jnp
from jax.experimental import pallas as pl

_S = 4
_V = 32
_BS = 8
_D = 1024
_HI = jax.lax.Precision.HIGHEST


def _prep_kernel(fg_ref, w1_ref, b1_ref, ds_ref, dt1_ref, dt2_ref, dt3_ref,
                 fm0_ref, s2s_ref, s2t1_ref, s2t2_ref, s2t3_ref):
    fm0_ref[...] = (
        jnp.dot(fg_ref[...], w1_ref[...], precision=_HI,
                preferred_element_type=jnp.float32) + b1_ref[...]
    )
    row = jax.lax.broadcasted_iota(jnp.int32, (_V, _D), 0)
    col = jax.lax.broadcasted_iota(jnp.int32, (_V, _D), 1)
    blk = (col // _V == row).astype(jnp.float32)

    def s2(dref):
        d = dref[...]
        cn = jnp.sqrt(jnp.sum(d * d, axis=0, keepdims=True))
        bs = jnp.dot(blk, d, precision=_HI, preferred_element_type=jnp.float32)
        return bs / jnp.maximum(cn, 1e-12)

    s2s_ref[...] = s2(ds_ref)
    s2t1_ref[...] = s2(dt1_ref)
    s2t2_ref[...] = s2(dt2_ref)
    s2t3_ref[...] = s2(dt3_ref)


def _decode_kernel(u_ref, s2s_ref, s2t1_ref, s2t2_ref, s2t3_ref,
                   wt1_ref, bt1_ref, wt2_ref, bt2_ref, wt3_ref, bt3_ref,
                   out_ref):
    u = u_ref[0]
    rr = jax.lax.broadcasted_iota(jnp.int32, (_V * _V, _V), 0)
    cc = jax.lax.broadcasted_iota(jnp.int32, (_V * _V, _V), 1)
    sel_v = (rr // _V == cc).astype(jnp.float32)
    sel_j = (rr % _V == cc).astype(jnp.float32)

    def rows(x):
        return jnp.dot(sel_j, x, precision=_HI,
                       preferred_element_type=jnp.float32)

    uv = jnp.dot(sel_v, u, precision=_HI,
                 preferred_element_type=jnp.float32)
    dif = rows(u) - uv
    d2r = jnp.sum(dif * dif, axis=1, keepdims=True) * float(_V)
    inv_norm = 1.0 / jnp.maximum(jnp.sqrt(d2r), 1e-12)

    inner = jax.lax.dot_general(u, u, (((1,), (1,)), ((), ())),
                                precision=_HI,
                                preferred_element_type=jnp.float32)
    qc = jnp.sum(u * u, axis=1, keepdims=True)
    i0 = jax.lax.broadcasted_iota(jnp.int32, (_V, _V), 0)
    i1 = jax.lax.broadcasted_iota(jnp.int32, (_V, _V), 1)
    qr = jnp.sum(jnp.where(i0 == i1, inner, 0.0), axis=0,
                 keepdims=True)
    d2m = (qc + qr) - 2.0 * inner
    dmin = jnp.min(d2m, axis=1, keepdims=True)
    nearest = jnp.min(jnp.where(d2m <= dmin, i1, 2 ** 30), axis=1,
                      keepdims=True)
    nrows = jnp.dot(sel_v, nearest.astype(jnp.float32), precision=_HI,
                    preferred_element_type=jnp.float32)
    jcol = (jax.lax.broadcasted_iota(jnp.int32, (_V * _V, 1), 0)
            % _V).astype(jnp.float32)
    negmask = jnp.where(jcol == nrows, -jnp.inf, 0.0)

    def combine(s2_ref, oc, support):
        th = jax.nn.relu(jnp.dot(dif, s2_ref[...], precision=_HI,
                                 preferred_element_type=jnp.float32)
                         * inv_norm)
        if support is not None:
            th = th * rows(support)
        th = th + negmask
        m = jnp.max(th.reshape(_V, _V, _S * oc), axis=1)
        acc = m[:, :oc]
        for s in range(1, _S):
            acc = acc + m[:, s * oc:(s + 1) * oc]
        return acc

    fm1 = jax.nn.relu(combine(s2s_ref, 32, None))
    fo = jnp.dot(fm1, wt1_ref[...], precision=_HI,
                 preferred_element_type=jnp.float32) + bt1_ref[...]
    fm2 = jax.nn.relu(fo[:, :32] + combine(s2t1_ref, 32, fo[:, 32:]))
    fo = jnp.dot(fm2, wt2_ref[...], precision=_HI,
                 preferred_element_type=jnp.float32) + bt2_ref[...]
    fm4 = jax.nn.relu(fo[:, :16] + combine(s2t2_ref, 16, fo[:, 16:]))
    fo = jnp.dot(fm4, wt3_ref[...], precision=_HI,
                 preferred_element_type=jnp.float32) + bt3_ref[...]
    out_ref[0] = fo[:, :3] + combine(s2t3_ref, 3, fo[:, 3:])


def kernel(feature_global, W1, b1, dir_s, w_t1, b_t1, dir_t1,
           w_t2, b_t2, dir_t2, w_t3, b_t3, dir_t3):
    f32 = jnp.float32
    fm0, s2s, s2t1, s2t2, s2t3 = pl.pallas_call(
        _prep_kernel,
        out_shape=(
            jax.ShapeDtypeStruct((_BS, _D), f32),
            jax.ShapeDtypeStruct((_V, _S * 32), f32),
            jax.ShapeDtypeStruct((_V, _S * 32), f32),
            jax.ShapeDtypeStruct((_V, _S * 16), f32),
            jax.ShapeDtypeStruct((_V, _S * 3), f32),
        ),
    )(feature_global, W1, b1.reshape(1, _D), dir_s, dir_t1, dir_t2, dir_t3)

    u = fm0.reshape(_BS, _V, _V)
    out = pl.pallas_call(
        _decode_kernel,
        grid=(_BS,),
        in_specs=[
            pl.BlockSpec((1, _V, _V), lambda b: (b, 0, 0)),
            pl.BlockSpec((_V, _S * 32), lambda b: (0, 0)),
            pl.BlockSpec((_V, _S * 32), lambda b: (0, 0)),
            pl.BlockSpec((_V, _S * 16), lambda b: (0, 0)),
            pl.BlockSpec((_V, _S * 3), lambda b: (0, 0)),
            pl.BlockSpec((32, 160), lambda b: (0, 0)),
            pl.BlockSpec((1, 160), lambda b: (0, 0)),
            pl.BlockSpec((32, 80), lambda b: (0, 0)),
            pl.BlockSpec((1, 80), lambda b: (0, 0)),
            pl.BlockSpec((16, 15), lambda b: (0, 0)),
            pl.BlockSpec((1, 15), lambda b: (0, 0)),
        ],
        out_specs=pl.BlockSpec((1, _V, 3), lambda b: (b, 0, 0)),
        out_shape=jax.ShapeDtypeStruct((_BS, _V, 3), f32),
    )(u, s2s, s2t1, s2t2, s2t3,
      w_t1, b_t1.reshape(1, 160), w_t2, b_t2.reshape(1, 80),
      w_t3, b_t3.reshape(1, 15))
    return out


# speedup vs baseline: 17.7195x; 2.6681x over previous
import jax
import jax.numpy as jnp
from jax.experimental import pallas as pl

_S = 4
_V = 32
_BS = 8
_D = 1024
_P = _BS * _V
_R = _P * _V
_HI = jax.lax.Precision.HIGHEST


def _prep_kernel(fg_ref, w1_ref, b1_ref, ds_ref, dt1_ref, dt2_ref, dt3_ref,
                 fm0_ref, s2s_ref, s2t1_ref, s2t2_ref, s2t3_ref):
    fm0_ref[...] = (
        jnp.dot(fg_ref[...], w1_ref[...], precision=_HI,
                preferred_element_type=jnp.float32) + b1_ref[...]
    )
    row = jax.lax.broadcasted_iota(jnp.int32, (_V, _D), 0)
    col = jax.lax.broadcasted_iota(jnp.int32, (_V, _D), 1)
    blk = (col // _V == row).astype(jnp.float32)

    def s2(dref):
        d = dref[...]
        cn = jnp.sqrt(jnp.sum(d * d, axis=0, keepdims=True))
        bs = jnp.dot(blk, d, precision=_HI, preferred_element_type=jnp.float32)
        return bs / jnp.maximum(cn, 1e-12)

    s2s_ref[...] = s2(ds_ref)
    s2t1_ref[...] = s2(dt1_ref)
    s2t2_ref[...] = s2(dt2_ref)
    s2t3_ref[...] = s2(dt3_ref)


def _pairs(x):
    c = x.shape[-1]
    x4 = jnp.broadcast_to(x.reshape(_BS, 1, _V, c), (_BS, _V, _V, c))
    return x4.reshape(_R, c)


def _centers(x):
    c = x.shape[-1]
    x3 = jnp.broadcast_to(x.reshape(_P, 1, c), (_P, _V, c))
    return x3.reshape(_R, c)


def _decode_kernel(u_ref, s2s_ref, s2t1_ref, s2t2_ref, s2t3_ref,
                   wt1_ref, bt1_ref, wt2_ref, bt2_ref, wt3_ref, bt3_ref,
                   out_ref):
    u = u_ref[...]
    dif = _pairs(u) - _centers(u)
    d2r = jnp.sum(dif * dif, axis=1, keepdims=True) * float(_V)
    inv_norm = 1.0 / jnp.maximum(jnp.sqrt(d2r), 1e-12)

    d3 = d2r.reshape(_P, _V, 1)
    dmin = jnp.min(d3, axis=1, keepdims=True)
    ji3 = jax.lax.broadcasted_iota(jnp.int32, (_P, _V, 1), 1)
    near = jnp.min(jnp.where(d3 <= dmin, ji3, 2 ** 30), axis=1,
                   keepdims=True)
    exclude = ji3 == jnp.broadcast_to(near, (_P, _V, 1))
    negmask = jnp.where(exclude, -jnp.inf, 0.0).reshape(_R, 1)

    def combine(s2_ref, oc, support):
        g = jnp.dot(u, s2_ref[...], precision=_HI,
                    preferred_element_type=jnp.float32)
        th = jax.nn.relu((_pairs(g) - _centers(g)) * inv_norm)
        if support is not None:
            th = th * _pairs(support)
        th = th + negmask
        m = jnp.max(th.reshape(_P, _V, _S * oc), axis=1)
        acc = m[:, :oc]
        for s in range(1, _S):
            acc = acc + m[:, s * oc:(s + 1) * oc]
        return acc

    fm1 = jax.nn.relu(combine(s2s_ref, 32, None))
    fo = jnp.dot(fm1, wt1_ref[...], precision=_HI,
                 preferred_element_type=jnp.float32) + bt1_ref[...]
    fm2 = jax.nn.relu(fo[:, :32] + combine(s2t1_ref, 32, fo[:, 32:]))
    fo = jnp.dot(fm2, wt2_ref[...], precision=_HI,
                 preferred_element_type=jnp.float32) + bt2_ref[...]
    fm4 = jax.nn.relu(fo[:, :16] + combine(s2t2_ref, 16, fo[:, 16:]))
    fo = jnp.dot(fm4, wt3_ref[...], precision=_HI,
                 preferred_element_type=jnp.float32) + bt3_ref[...]
    out_ref[...] = fo[:, :3] + combine(s2t3_ref, 3, fo[:, 3:])


def kernel(feature_global, W1, b1, dir_s, w_t1, b_t1, dir_t1,
           w_t2, b_t2, dir_t2, w_t3, b_t3, dir_t3):
    f32 = jnp.float32
    fm0, s2s, s2t1, s2t2, s2t3 = pl.pallas_call(
        _prep_kernel,
        out_shape=(
            jax.ShapeDtypeStruct((_BS, _D), f32),
            jax.ShapeDtypeStruct((_V, _S * 32), f32),
            jax.ShapeDtypeStruct((_V, _S * 32), f32),
            jax.ShapeDtypeStruct((_V, _S * 16), f32),
            jax.ShapeDtypeStruct((_V, _S * 3), f32),
        ),
    )(feature_global, W1, b1.reshape(1, _D), dir_s, dir_t1, dir_t2, dir_t3)

    u_all = fm0.reshape(_P, _V)
    out = pl.pallas_call(
        _decode_kernel,
        out_shape=jax.ShapeDtypeStruct((_P, 3), f32),
    )(u_all, s2s, s2t1, s2t2, s2t3,
      w_t1, b_t1.reshape(1, 160), w_t2, b_t2.reshape(1, 80),
      w_t3, b_t3.reshape(1, 15))
    return out.reshape(_BS, _V, 3)


# speedup vs baseline: 20.7338x; 1.1701x over previous
import jax
import jax.numpy as jnp
from jax.experimental import pallas as pl

_S = 4
_V = 32
_BS = 8
_D = 1024
_P = _V * _BS
_R = _V * _P
_HI = jax.lax.Precision.HIGHEST


def _prep_kernel(fg_ref, w1_ref, b1_ref, ds_ref, dt1_ref, dt2_ref, dt3_ref,
                 fm0_ref, s2s_ref, s2t1_ref, s2t2_ref, s2t3_ref):
    fm0_ref[...] = (
        jnp.dot(fg_ref[...], w1_ref[...], precision=_HI,
                preferred_element_type=jnp.float32) + b1_ref[...]
    )
    row = jax.lax.broadcasted_iota(jnp.int32, (_V, _D), 0)
    col = jax.lax.broadcasted_iota(jnp.int32, (_V, _D), 1)
    blk = (col // _V == row).astype(jnp.float32)

    def s2(dref):
        d = dref[...]
        cn = jnp.sqrt(jnp.sum(d * d, axis=0, keepdims=True))
        bs = jnp.dot(blk, d, precision=_HI,
                     preferred_element_type=jnp.float32)
        return bs / jnp.maximum(cn, 1e-12)

    s2s_ref[...] = s2(ds_ref)
    s2t1_ref[...] = s2(dt1_ref)
    s2t2_ref[...] = s2(dt2_ref)
    s2t3_ref[...] = s2(dt3_ref)


def _decode_kernel(u_ref, s2s_ref, s2t1_ref, s2t2_ref, s2t3_ref,
                   wt1_ref, bt1_ref, wt2_ref, bt2_ref, wt3_ref, bt3_ref,
                   out_ref):
    u = u_ref[...]

    def pairs(x):
        c = x.shape[-1]
        x4 = jnp.broadcast_to(x.reshape(1, _V, _BS, c), (_V, _V, _BS, c))
        return x4.reshape(_R, c)

    def centers(x):
        c = x.shape[-1]
        x4 = jnp.broadcast_to(x.reshape(_V, 1, _BS, c), (_V, _V, _BS, c))
        return x4.reshape(_R, c)

    dif = pairs(u) - centers(u)
    d2r = jnp.sum(dif * dif, axis=1, keepdims=True) * float(_V)
    inv_norm = jax.lax.rsqrt(jnp.maximum(d2r, 1e-24))

    d4 = d2r.reshape(_V, _V, _BS, 1)
    dmin = jnp.min(d4, axis=1, keepdims=True)
    ji4 = jax.lax.broadcasted_iota(jnp.int32, (_V, _V, _BS, 1), 1)
    near = jnp.min(jnp.where(d4 <= jnp.broadcast_to(dmin, d4.shape),
                             ji4, 2 ** 30),
                   axis=1, keepdims=True)
    exclude = ji4 == jnp.broadcast_to(near, d4.shape)
    negmask = jnp.where(exclude, -jnp.inf, 0.0).reshape(_R, 1)
    inv_norm_z = jnp.where(exclude.reshape(_R, 1), 0.0, inv_norm)

    def combine(s2_ref, oc, support):
        g = jnp.dot(u, s2_ref[...], precision=_HI,
                    preferred_element_type=jnp.float32)
        if support is None:
            th = jax.nn.relu((pairs(g) - centers(g)) * inv_norm_z)
        else:
            th = jax.nn.relu((pairs(g) - centers(g)) * inv_norm)
            th = th * pairs(support)
            th = th + negmask
        m = jnp.max(th.reshape(_V, _V, _BS, _S * oc), axis=1)
        mm = m.reshape(_P, _S * oc)
        acc = mm[:, :oc]
        for s in range(1, _S):
            acc = acc + mm[:, s * oc:(s + 1) * oc]
        return acc

    fm1 = jax.nn.relu(combine(s2s_ref, 32, None))
    fo = jnp.dot(fm1, wt1_ref[...], precision=_HI,
                 preferred_element_type=jnp.float32) + bt1_ref[...]
    fm2 = jax.nn.relu(fo[:, :32] + combine(s2t1_ref, 32, fo[:, 32:]))
    fo = jnp.dot(fm2, wt2_ref[...], precision=_HI,
                 preferred_element_type=jnp.float32) + bt2_ref[...]
    fm4 = jax.nn.relu(fo[:, :16] + combine(s2t2_ref, 16, fo[:, 16:]))
    fo = jnp.dot(fm4, wt3_ref[...], precision=_HI,
                 preferred_element_type=jnp.float32) + bt3_ref[...]
    out_ref[...] = fo[:, :3] + combine(s2t3_ref, 3, fo[:, 3:])


def kernel(feature_global, W1, b1, dir_s, w_t1, b_t1, dir_t1,
           w_t2, b_t2, dir_t2, w_t3, b_t3, dir_t3):
    f32 = jnp.float32
    fm0, s2s, s2t1, s2t2, s2t3 = pl.pallas_call(
        _prep_kernel,
        out_shape=(
            jax.ShapeDtypeStruct((_BS, _D), f32),
            jax.ShapeDtypeStruct((_V, _S * 32), f32),
            jax.ShapeDtypeStruct((_V, _S * 32), f32),
            jax.ShapeDtypeStruct((_V, _S * 16), f32),
            jax.ShapeDtypeStruct((_V, _S * 3), f32),
        ),
    )(feature_global, W1, b1.reshape(1, _D), dir_s, dir_t1, dir_t2, dir_t3)

    u_all = fm0.reshape(_BS, _V, _V).transpose(1, 0, 2).reshape(_P, _V)

    out = pl.pallas_call(
        _decode_kernel,
        out_shape=jax.ShapeDtypeStruct((_P, 3), f32),
    )(u_all, s2s, s2t1, s2t2, s2t3,
      w_t1, b_t1.reshape(1, 160), w_t2, b_t2.reshape(1, 80),
      w_t3, b_t3.reshape(1, 15))
    return out.reshape(_V, _BS, 3).transpose(1, 0, 2)


# speedup vs baseline: 23.2300x; 1.1204x over previous
import jax
import jax.numpy as jnp
from jax.experimental import pallas as pl

_S = 4
_V = 32
_BS = 8
_D = 1024
_P = _V * _BS
_R = _V * _P
_HI = jax.lax.Precision.HIGHEST


def _fused_kernel(fg_ref, w1_ref, b1_ref, ds_ref, dt1_ref, dt2_ref, dt3_ref,
                  wt1_ref, bt1_ref, wt2_ref, bt2_ref, wt3_ref, bt3_ref,
                  out_ref):
    fm0 = (
        jnp.dot(fg_ref[...], w1_ref[...], precision=_HI,
                preferred_element_type=jnp.float32) + b1_ref[...]
    )
    u = jnp.swapaxes(fm0.transpose(1, 0).reshape(_V, _V, _BS), 1, 2)
    u = u.reshape(_P, _V)

    row = jax.lax.broadcasted_iota(jnp.int32, (_V, _D), 0)
    col = jax.lax.broadcasted_iota(jnp.int32, (_V, _D), 1)
    blk = (col // _V == row).astype(jnp.float32)

    def s2(dref):
        d = dref[...]
        cn = jnp.sqrt(jnp.sum(d * d, axis=0, keepdims=True))
        bs = jnp.dot(blk, d, precision=_HI,
                     preferred_element_type=jnp.float32)
        return bs / jnp.maximum(cn, 1e-12)

    def pairs(x):
        c = x.shape[-1]
        x4 = jnp.broadcast_to(x.reshape(1, _V, _BS, c), (_V, _V, _BS, c))
        return x4.reshape(_R, c)

    def centers(x):
        c = x.shape[-1]
        x4 = jnp.broadcast_to(x.reshape(_V, 1, _BS, c), (_V, _V, _BS, c))
        return x4.reshape(_R, c)

    dif = pairs(u) - centers(u)
    d2r = jnp.sum(dif * dif, axis=1, keepdims=True) * float(_V)
    inv_norm = jax.lax.rsqrt(jnp.maximum(d2r, 1e-24))

    d4 = d2r.reshape(_V, _V, _BS, 1)
    dmin = jnp.min(d4, axis=1, keepdims=True)
    ji4 = jax.lax.broadcasted_iota(jnp.int32, (_V, _V, _BS, 1), 1)
    near = jnp.min(jnp.where(d4 <= jnp.broadcast_to(dmin, d4.shape),
                             ji4, 2 ** 30),
                   axis=1, keepdims=True)
    exclude = ji4 == jnp.broadcast_to(near, d4.shape)
    negmask = jnp.where(exclude, -jnp.inf, 0.0).reshape(_R, 1)
    inv_norm_z = jnp.where(exclude.reshape(_R, 1), 0.0, inv_norm)

    def combine(s2m, oc, support):
        g = jnp.dot(u, s2m, precision=_HI,
                    preferred_element_type=jnp.float32)
        if support is None:
            th = jax.nn.relu((pairs(g) - centers(g)) * inv_norm_z)
        else:
            th = jax.nn.relu((pairs(g) - centers(g)) * inv_norm)
            th = th * pairs(support)
            th = th + negmask
        m = jnp.max(th.reshape(_V, _V, _BS, _S * oc), axis=1)
        mm = m.reshape(_P, _S * oc)
        acc = mm[:, :oc]
        for s in range(1, _S):
            acc = acc + mm[:, s * oc:(s + 1) * oc]
        return acc

    fm1 = jax.nn.relu(combine(s2(ds_ref), 32, None))
    fo = jnp.dot(fm1, wt1_ref[...], precision=_HI,
                 preferred_element_type=jnp.float32) + bt1_ref[...]
    fm2 = jax.nn.relu(fo[:, :32] + combine(s2(dt1_ref), 32, fo[:, 32:]))
    fo = jnp.dot(fm2, wt2_ref[...], precision=_HI,
                 preferred_element_type=jnp.float32) + bt2_ref[...]
    fm4 = jax.nn.relu(fo[:, :16] + combine(s2(dt2_ref), 16, fo[:, 16:]))
    fo = jnp.dot(fm4, wt3_ref[...], precision=_HI,
                 preferred_element_type=jnp.float32) + bt3_ref[...]
    res = fo[:, :3] + combine(s2(dt3_ref), 3, fo[:, 3:])
    out_ref[...] = jnp.swapaxes(res.reshape(_V, _BS, 3), 0, 1)


def kernel(feature_global, W1, b1, dir_s, w_t1, b_t1, dir_t1,
           w_t2, b_t2, dir_t2, w_t3, b_t3, dir_t3):
    f32 = jnp.float32
    return pl.pallas_call(
        _fused_kernel,
        out_shape=jax.ShapeDtypeStruct((_BS, _V, 3), f32),
    )(feature_global, W1, b1.reshape(1, _D), dir_s, dir_t1, dir_t2, dir_t3,
      w_t1, b_t1.reshape(1, 160), w_t2, b_t2.reshape(1, 80),
      w_t3, b_t3.reshape(1, 15))


# speedup vs baseline: 23.7627x; 1.0229x over previous
import jax
import jax.numpy as jnp
from jax.experimental import pallas as pl

_S = 4
_V = 32
_BS = 8
_D = 1024
_P = _V * _BS
_R = _V * _P
_HI = jax.lax.Precision.HIGHEST


def _fused_kernel(fg_ref, w1_ref, b1_ref, ds_ref, dt1_ref, dt2_ref, dt3_ref,
                  wt1_ref, bt1_ref, wt2_ref, bt2_ref, wt3_ref, bt3_ref,
                  out_ref):
    fm0 = (
        jnp.dot(fg_ref[...], w1_ref[...],
                preferred_element_type=jnp.float32) + b1_ref[...]
    )
    u = jnp.swapaxes(fm0.transpose(1, 0).reshape(_V, _V, _BS), 1, 2)
    u = u.reshape(_P, _V)

    row = jax.lax.broadcasted_iota(jnp.int32, (_V, _D), 0)
    col = jax.lax.broadcasted_iota(jnp.int32, (_V, _D), 1)
    blk = (col // _V == row).astype(jnp.float32)

    ones_row = jnp.full((1, _D), 1.0, jnp.float32)

    def s2(dref):
        d = dref[...]
        cn = jnp.sqrt(jnp.dot(ones_row, d * d, precision=_HI,
                              preferred_element_type=jnp.float32))
        bs = jnp.dot(blk, d, preferred_element_type=jnp.float32)
        return bs / jnp.maximum(cn, 1e-12)

    def pairs(x):
        c = x.shape[-1]
        x4 = jnp.broadcast_to(x.reshape(1, _V, _BS, c), (_V, _V, _BS, c))
        return x4.reshape(_R, c)

    def centers(x):
        c = x.shape[-1]
        x4 = jnp.broadcast_to(x.reshape(_V, 1, _BS, c), (_V, _V, _BS, c))
        return x4.reshape(_R, c)

    dif = pairs(u) - centers(u)
    d2r = jnp.sum(dif * dif, axis=1, keepdims=True) * float(_V)
    inv_norm = jax.lax.rsqrt(jnp.maximum(d2r, 1e-24))

    d4 = d2r.reshape(_V, _V, _BS, 1)
    dmin = jnp.min(d4, axis=1, keepdims=True)
    ji4 = jax.lax.broadcasted_iota(jnp.int32, (_V, _V, _BS, 1), 1)
    near = jnp.min(jnp.where(d4 <= jnp.broadcast_to(dmin, d4.shape),
                             ji4, 2 ** 30),
                   axis=1, keepdims=True)
    exclude = ji4 == jnp.broadcast_to(near, d4.shape)
    negmask = jnp.where(exclude, -jnp.inf, 0.0).reshape(_R, 1)
    inv_norm_z = jnp.where(exclude.reshape(_R, 1), 0.0, inv_norm)

    def combine(s2m, oc, support):
        g = jnp.dot(u, s2m, precision=_HI,
                    preferred_element_type=jnp.float32)
        if support is None:
            th = jax.nn.relu((pairs(g) - centers(g)) * inv_norm_z)
        else:
            th = jax.nn.relu((pairs(g) - centers(g)) * inv_norm)
            th = th * pairs(support)
            th = th + negmask
        m = jnp.max(th.reshape(_V, _V, _BS, _S * oc), axis=1)
        mm = m.reshape(_P, _S * oc)
        acc = mm[:, :oc]
        for s in range(1, _S):
            acc = acc + mm[:, s * oc:(s + 1) * oc]
        return acc

    fm1 = jax.nn.relu(combine(s2(ds_ref), 32, None))
    fo = jnp.dot(fm1, wt1_ref[...], precision=_HI,
                 preferred_element_type=jnp.float32) + bt1_ref[...]
    fm2 = jax.nn.relu(fo[:, :32] + combine(s2(dt1_ref), 32, fo[:, 32:]))
    fo = jnp.dot(fm2, wt2_ref[...], precision=_HI,
                 preferred_element_type=jnp.float32) + bt2_ref[...]
    fm4 = jax.nn.relu(fo[:, :16] + combine(s2(dt2_ref), 16, fo[:, 16:]))
    fo = jnp.dot(fm4, wt3_ref[...], precision=_HI,
                 preferred_element_type=jnp.float32) + bt3_ref[...]
    res = fo[:, :3] + combine(s2(dt3_ref), 3, fo[:, 3:])
    out_ref[...] = jnp.swapaxes(res.reshape(_V, _BS, 3), 0, 1)


def kernel(feature_global, W1, b1, dir_s, w_t1, b_t1, dir_t1,
           w_t2, b_t2, dir_t2, w_t3, b_t3, dir_t3):
    f32 = jnp.float32
    return pl.pallas_call(
        _fused_kernel,
        out_shape=jax.ShapeDtypeStruct((_BS, _V, 3), f32),
    )(feature_global, W1, b1.reshape(1, _D), dir_s, dir_t1, dir_t2, dir_t3,
      w_t1, b_t1.reshape(1, 160), w_t2, b_t2.reshape(1, 80),
      w_t3, b_t3.reshape(1, 15))


# speedup vs baseline: 25.8855x; 1.0893x over previous
import jax
import jax.numpy as jnp
from jax.experimental import pallas as pl

_S = 4
_V = 32
_BS = 8
_D = 1024
_P = _V * _BS
_R = _V * _P
_HI = jax.lax.Precision.HIGHEST


def _fused_kernel(fg_ref, w1_ref, b1_ref, ds_ref, dt1_ref, dt2_ref, dt3_ref,
                  wt1_ref, bt1_ref, wt2_ref, bt2_ref, wt3_ref, bt3_ref,
                  out_ref):
    fm0 = (
        jnp.dot(fg_ref[...], w1_ref[...],
                preferred_element_type=jnp.float32) + b1_ref[...]
    )
    u = jnp.swapaxes(fm0.transpose(1, 0).reshape(_V, _V, _BS), 1, 2)
    u = u.reshape(_P, _V)

    row = jax.lax.broadcasted_iota(jnp.int32, (_V, _D), 0)
    col = jax.lax.broadcasted_iota(jnp.int32, (_V, _D), 1)
    blk = (col // _V == row).astype(jnp.float32)

    ones_row = jnp.full((1, _D), 1.0, jnp.float32)

    def s2(dref):
        d = dref[...]
        cn = jnp.sqrt(jnp.dot(ones_row, d * d, precision=_HI,
                              preferred_element_type=jnp.float32))
        bs = jnp.dot(blk, d, preferred_element_type=jnp.float32)
        return bs / jnp.maximum(cn, 1e-12)

    def pairs(x):
        c = x.shape[-1]
        x4 = jnp.broadcast_to(x.reshape(1, _V, _BS, c), (_V, _V, _BS, c))
        return x4.reshape(_R, c)

    def centers(x):
        c = x.shape[-1]
        x4 = jnp.broadcast_to(x.reshape(_V, 1, _BS, c), (_V, _V, _BS, c))
        return x4.reshape(_R, c)

    dif = pairs(u) - centers(u)
    d2r = jnp.sum(dif * dif, axis=1, keepdims=True) * float(_V)
    inv_norm = jax.lax.rsqrt(jnp.maximum(d2r, 1e-24))

    d4 = d2r.reshape(_V, _V, _BS, 1)
    dmin = jnp.min(d4, axis=1, keepdims=True)
    ji4 = jax.lax.broadcasted_iota(jnp.int32, (_V, _V, _BS, 1), 1)
    near = jnp.min(jnp.where(d4 <= jnp.broadcast_to(dmin, d4.shape),
                             ji4, 2 ** 30),
                   axis=1, keepdims=True)
    exclude = ji4 == jnp.broadcast_to(near, d4.shape)
    negmask = jnp.where(exclude, -jnp.inf, 0.0).reshape(_R, 1)
    inv_norm_z = jnp.where(exclude.reshape(_R, 1), 0.0, inv_norm)

    def combine(s2m, oc, support):
        g = jnp.dot(u, s2m,
                    preferred_element_type=jnp.float32)
        if support is None:
            th = jax.nn.relu((pairs(g) - centers(g)) * inv_norm_z)
        else:
            th = jax.nn.relu((pairs(g) - centers(g)) * inv_norm)
            th = th * pairs(support)
            th = th + negmask
        m = jnp.max(th.reshape(_V, _V, _BS, _S * oc), axis=1)
        mm = m.reshape(_P, _S * oc)
        acc = mm[:, :oc]
        for s in range(1, _S):
            acc = acc + mm[:, s * oc:(s + 1) * oc]
        return acc

    fm1 = jax.nn.relu(combine(s2(ds_ref), 32, None))
    fo = jnp.dot(fm1, wt1_ref[...],
                 preferred_element_type=jnp.float32) + bt1_ref[...]
    fm2 = jax.nn.relu(fo[:, :32] + combine(s2(dt1_ref), 32, fo[:, 32:]))
    fo = jnp.dot(fm2, wt2_ref[...],
                 preferred_element_type=jnp.float32) + bt2_ref[...]
    fm4 = jax.nn.relu(fo[:, :16] + combine(s2(dt2_ref), 16, fo[:, 16:]))
    fo = jnp.dot(fm4, wt3_ref[...],
                 preferred_element_type=jnp.float32) + bt3_ref[...]
    res = fo[:, :3] + combine(s2(dt3_ref), 3, fo[:, 3:])
    out_ref[...] = jnp.swapaxes(res.reshape(_V, _BS, 3), 0, 1)


def kernel(feature_global, W1, b1, dir_s, w_t1, b_t1, dir_t1,
           w_t2, b_t2, dir_t2, w_t3, b_t3, dir_t3):
    f32 = jnp.float32
    return pl.pallas_call(
        _fused_kernel,
        out_shape=jax.ShapeDtypeStruct((_BS, _V, 3), f32),
    )(feature_global, W1, b1.reshape(1, _D), dir_s, dir_t1, dir_t2, dir_t3,
      w_t1, b_t1.reshape(1, 160), w_t2, b_t2.reshape(1, 80),
      w_t3, b_t3.reshape(1, 15))


# speedup vs baseline: 26.8552x; 1.0375x over previous
import jax
import jax.numpy as jnp
from jax.experimental import pallas as pl

_S = 4
_V = 32
_BS = 8
_D = 1024
_P = _V * _BS
_R = _V * _P
_HI = jax.lax.Precision.HIGHEST


def _fused_kernel(fg_ref, w1_ref, b1_ref, ds_ref, dt1_ref, dt2_ref, dt3_ref,
                  wt1_ref, bt1_ref, wt2_ref, bt2_ref, wt3_ref, bt3_ref,
                  out_ref):
    fm0 = (
        jnp.dot(fg_ref[...], w1_ref[...],
                preferred_element_type=jnp.float32) + b1_ref[...]
    )
    u = jnp.swapaxes(fm0.transpose(1, 0).reshape(_V, _V, _BS), 1, 2)
    u = u.reshape(_P, _V)

    row = jax.lax.broadcasted_iota(jnp.int32, (_V, _D), 0)
    col = jax.lax.broadcasted_iota(jnp.int32, (_V, _D), 1)
    blk = (col // _V == row).astype(jnp.float32)

    ones_row = jnp.full((1, _D), 1.0, jnp.float32)

    def s2(dref):
        d = dref[...]
        cn = jnp.sqrt(jnp.dot(ones_row, d * d,
                              preferred_element_type=jnp.float32))
        bs = jnp.dot(blk, d, preferred_element_type=jnp.float32)
        return bs / jnp.maximum(cn, 1e-12)

    def pairs(x):
        c = x.shape[-1]
        x4 = jnp.broadcast_to(x.reshape(1, _V, _BS, c), (_V, _V, _BS, c))
        return x4.reshape(_R, c)

    def centers(x):
        c = x.shape[-1]
        x4 = jnp.broadcast_to(x.reshape(_V, 1, _BS, c), (_V, _V, _BS, c))
        return x4.reshape(_R, c)

    dif = pairs(u) - centers(u)
    d2r = jnp.sum(dif * dif, axis=1, keepdims=True) * float(_V)
    inv_norm = jax.lax.rsqrt(jnp.maximum(d2r, 1e-24))

    d4 = d2r.reshape(_V, _V, _BS, 1)
    dmin = jnp.min(d4, axis=1, keepdims=True)
    ji4 = jax.lax.broadcasted_iota(jnp.int32, (_V, _V, _BS, 1), 1)
    near = jnp.min(jnp.where(d4 <= jnp.broadcast_to(dmin, d4.shape),
                             ji4, 2 ** 30),
                   axis=1, keepdims=True)
    exclude = ji4 == jnp.broadcast_to(near, d4.shape)
    negmask = jnp.where(exclude, -jnp.inf, 0.0).reshape(_R, 1)
    inv_norm_z = jnp.where(exclude.reshape(_R, 1), 0.0, inv_norm)

    def combine(s2m, oc, support):
        g = jnp.dot(u, s2m,
                    preferred_element_type=jnp.float32)
        if support is None:
            th = jax.nn.relu((pairs(g) - centers(g)) * inv_norm_z)
        else:
            th = jax.nn.relu((pairs(g) - centers(g)) * inv_norm)
            th = th * pairs(support)
            th = th + negmask
        m = jnp.max(th.reshape(_V, _V, _BS, _S * oc), axis=1)
        mm = m.reshape(_P, _S * oc)
        acc = mm[:, :oc]
        for s in range(1, _S):
            acc = acc + mm[:, s * oc:(s + 1) * oc]
        return acc

    fm1 = jax.nn.relu(combine(s2(ds_ref), 32, None))
    fo = jnp.dot(fm1, wt1_ref[...],
                 preferred_element_type=jnp.float32) + bt1_ref[...]
    fm2 = jax.nn.relu(fo[:, :32] + combine(s2(dt1_ref), 32, fo[:, 32:]))
    fo = jnp.dot(fm2, wt2_ref[...],
                 preferred_element_type=jnp.float32) + bt2_ref[...]
    fm4 = jax.nn.relu(fo[:, :16] + combine(s2(dt2_ref), 16, fo[:, 16:]))
    fo = jnp.dot(fm4, wt3_ref[...],
                 preferred_element_type=jnp.float32) + bt3_ref[...]
    res = fo[:, :3] + combine(s2(dt3_ref), 3, fo[:, 3:])
    out_ref[...] = jnp.swapaxes(res.reshape(_V, _BS, 3), 0, 1)


def kernel(feature_global, W1, b1, dir_s, w_t1, b_t1, dir_t1,
           w_t2, b_t2, dir_t2, w_t3, b_t3, dir_t3):
    f32 = jnp.float32
    return pl.pallas_call(
        _fused_kernel,
        out_shape=jax.ShapeDtypeStruct((_BS, _V, 3), f32),
    )(feature_global, W1, b1.reshape(1, _D), dir_s, dir_t1, dir_t2, dir_t3,
      w_t1, b_t1.reshape(1, 160), w_t2, b_t2.reshape(1, 80),
      w_t3, b_t3.reshape(1, 15))
